# baseline (speedup 1.0000x reference)
import jax
import jax.numpy as jnp
from jax import lax
from jax.experimental import pallas as pl
from jax.experimental.pallas import tpu as pltpu
from jax.experimental.pallas import tpu_sc as plsc

NN = 10000
D = 128
HD = 64
E = 320000
EPOS = 160000
NC = 2
NS = 16
NW = NC * NS
PERW = E // NW
PERT = E // NS
K = 80
NCHA = PERW // K
NCHB = PERT // K
NNP = 10240
RPT = NNP // NS


def _sc_mesh():
    return plsc.VectorSubcoreMesh(core_axis_name="c", subcore_axis_name="s",
                                  num_cores=NC, num_subcores=NS)


def _mul_rows(ra, rb, oa, ob):
    def rowfn(i, c2):
        for j in range(4):
            sl = pl.ds(j * 16, 16)
            oa[i, sl] = ra[i, sl] * rb[i, sl]
        for j in range(4, 8):
            sl = pl.ds(j * 16, 16)
            sl2 = pl.ds((j - 4) * 16, 16)
            ob[i, sl2] = ra[i, sl] * rb[i, sl]
        return c2

    lax.fori_loop(0, K, rowfn, 0)


def _phase_a(nfea, src, dst, hA, hB, sall, dall, ra0, rb0, oa0, ob0,
             ra1, rb1, oa1, ob1, sg0, sg1, sw0, sw1):
    wid = lax.axis_index("c") * NS + lax.axis_index("s")
    base = wid * PERW

    pltpu.sync_copy(src.at[pl.ds(base, PERW)], sall)
    pltpu.sync_copy(dst.at[pl.ds(base, PERW)], dall)

    def gathers(ci, ra, rb, sg):
        off = ci * K
        pltpu.async_copy(nfea.at[sall.at[pl.ds(off, K)]], ra, sg)
        pltpu.async_copy(nfea.at[dall.at[pl.ds(off, K)]], rb, sg)

    gathers(0, ra0, rb0, sg0)
    gathers(1, ra1, rb1, sg1)

    def slot(ci, first, ra, rb, oa, ob, sg, sw):
        oA_slice = hA.at[pl.ds(base + ci * K, K)]
        oB_slice = hB.at[pl.ds(base + ci * K, K)]
        gd = pltpu.make_async_copy(nfea.at[sall.at[pl.ds(0, K)]], ra, sg)
        gd.wait()
        gd.wait()
        if not first:
            pltpu.make_async_copy(oa, oA_slice, sw).wait()
            pltpu.make_async_copy(ob, oB_slice, sw).wait()
        _mul_rows(ra, rb, oa, ob)
        pltpu.async_copy(oa, oA_slice, sw)
        pltpu.async_copy(ob, oB_slice, sw)

        if isinstance(ci, int):
            if ci + 2 < NCHA:
                gathers(ci + 2, ra, rb, sg)
        else:
            @pl.when(ci + 2 < NCHA)
            def _():
                gathers(ci + 2, ra, rb, sg)

    def pair(g, carry):
        slot(2 * g, False, ra0, rb0, oa0, ob0, sg0, sw0)
        slot(2 * g + 1, False, ra1, rb1, oa1, ob1, sg1, sw1)
        return carry

    slot(0, True, ra0, rb0, oa0, ob0, sg0, sw0)
    slot(1, True, ra1, rb1, oa1, ob1, sg1, sw1)
    lax.fori_loop(1, 62, pair, 0)
    slot(124, False, ra0, rb0, oa0, ob0, sg0, sw0)
    pltpu.make_async_copy(oa0, hA.at[pl.ds(base, K)], sw0).wait()
    pltpu.make_async_copy(ob0, hB.at[pl.ds(base, K)], sw0).wait()
    pltpu.make_async_copy(oa1, hA.at[pl.ds(base, K)], sw1).wait()
    pltpu.make_async_copy(ob1, hB.at[pl.ds(base, K)], sw1).wait()


def _phase_b(hA, hB, src, dst, aggr_out, cnt_out, sall, db0, db1, rw0,
             rw1, obuf, zb, zc, aggr_sp, cnt_sp, si0, si1, sg0, sg1):
    cid = lax.axis_index("c")
    sid = lax.axis_index("s")
    base = sid * PERT

    zero16 = jnp.zeros((16,), jnp.float32)
    e0 = jnp.where(lax.iota(jnp.int32, 16) == 0, 1.0, 0.0).astype(jnp.float32)

    def zrow(i, c2):
        for j in range(4):
            zb[i, pl.ds(j * 16, 16)] = zero16
        return c2

    lax.fori_loop(0, 40, zrow, 0)

    def zcrow(i, c2):
        zc[i, :] = zero16
        return c2

    lax.fori_loop(0, 40, zcrow, 0)

    def orow(i, c2):
        obuf[i, :] = e0
        return c2

    lax.fori_loop(0, K, orow, 0)

    for j in range(16):
        pltpu.sync_copy(zb, aggr_sp.at[pl.ds(sid * RPT + j * 40, 40)])
        pltpu.sync_copy(zc, cnt_sp.at[pl.ds(sid * RPT + j * 40, 40)])

    pltpu.sync_copy(src.at[pl.ds(base, PERT)], sall)
    plsc.subcore_barrier()

    def issue(ci, db, rw, si, sg):
        off = ci * K
        pltpu.async_copy(dst.at[pl.ds(base + off, K)], db, si)

        @pl.when(cid == 0)
        def _():
            pltpu.async_copy(hA.at[sall.at[pl.ds(off, K)]], rw, sg)

        @pl.when(cid == 1)
        def _():
            pltpu.async_copy(hB.at[sall.at[pl.ds(off, K)]], rw, sg)

    issue(0, db0, rw0, si0, sg0)
    issue(1, db1, rw1, si1, sg1)

    def slot(ci, even, db, rw, si, sg):
        pltpu.make_async_copy(dst.at[pl.ds(base, K)], db, si).wait()
        pltpu.make_async_copy(hA.at[sall.at[pl.ds(0, K)]], rw, sg).wait()
        pltpu.sync_copy(rw, aggr_sp.at[db], add=True)

        @pl.when(cid == (0 if even else 1))
        def _():
            pltpu.sync_copy(obuf, cnt_sp.at[db], add=True)

        @pl.when(ci + 2 < NCHB)
        def _():
            issue(ci + 2, db, rw, si, sg)

    def pair(g, carry):
        slot(2 * g, True, db0, rw0, si0, sg0)
        slot(2 * g + 1, False, db1, rw1, si1, sg1)
        return carry

    lax.fori_loop(0, 125, pair, 0)
    plsc.subcore_barrier()

    pltpu.sync_copy(aggr_sp.at[pl.ds(sid * RPT, RPT)],
                    aggr_out.at[cid, pl.ds(sid * RPT, RPT)])
    pltpu.sync_copy(cnt_sp.at[pl.ds(sid * RPT, RPT)],
                    cnt_out.at[cid, pl.ds(sid * RPT, RPT)])


_T = 2000
_NT = E // _T
_NZT = NN // _T


def _tc_body(hA_ref, hB_ref, aA, aB, c0_ref, c1_ref, wl, bl, wr, w1, b1,
             w2, b2, w3, b3, out_ref, z_scr):
    i = pl.program_id(0)

    @pl.when(i < _NZT)
    def _():
        aggr = jnp.concatenate([aA[0], aB[0]], axis=1)
        cnt = c0_ref[0][:, 0:1] + c1_ref[0][:, 0:1]
        cnt1 = jnp.maximum(cnt, 1.0)
        z_scr[...] = jnp.dot(aggr / cnt1, wl[...],
                             preferred_element_type=jnp.float32)

    @pl.when(i >= _NZT)
    def _():
        z_scr[...] = jnp.zeros((_T, D), jnp.float32)

    h = jnp.concatenate([hA_ref[...], hB_ref[...]], axis=1)
    x = jnp.dot(h, wr[...], preferred_element_type=jnp.float32)
    x = x + bl[...] + z_scr[...]
    x = jnp.maximum(x, 0.0)
    x = jnp.maximum(jnp.dot(x, w1[...], preferred_element_type=jnp.float32)
                    + b1[...], 0.0)
    x = jnp.maximum(jnp.dot(x, w2[...], preferred_element_type=jnp.float32)
                    + b2[...], 0.0)
    lg = jnp.dot(x, w3[...], preferred_element_type=jnp.float32) + b3[...]
    m = jnp.max(lg, axis=-1, keepdims=True)
    lse = m + jnp.log(jnp.sum(jnp.exp(lg - m), axis=-1, keepdims=True))
    out_ref[...] = lg - lse


def kernel(rna_f, protein_f, all_edges, W_sage_l, b_sage, W_sage_r,
           W1, b1, W2, b2, W3, b3):
    n_fea = jnp.concatenate([rna_f, protein_f], axis=0)
    edges = all_edges[::2].T
    neg = jax.random.randint(jax.random.key(42), (2, EPOS), 0, NN,
                             edges.dtype)
    ei = jnp.concatenate([edges, neg], axis=1)
    src = ei[0]
    dst = ei[1]

    hA, hB = pl.kernel(
        _phase_a,
        out_type=[
            jax.ShapeDtypeStruct((E, HD), jnp.float32),
            jax.ShapeDtypeStruct((E, HD), jnp.float32),
        ],
        mesh=_sc_mesh(),
        compiler_params=pltpu.CompilerParams(use_tc_tiling_on_sc=False),
        scratch_types=[
            pltpu.VMEM((PERW,), jnp.int32),
            pltpu.VMEM((PERW,), jnp.int32),
            pltpu.VMEM((K, D), jnp.float32),
            pltpu.VMEM((K, D), jnp.float32),
            pltpu.VMEM((K, HD), jnp.float32),
            pltpu.VMEM((K, HD), jnp.float32),
            pltpu.VMEM((K, D), jnp.float32),
            pltpu.VMEM((K, D), jnp.float32),
            pltpu.VMEM((K, HD), jnp.float32),
            pltpu.VMEM((K, HD), jnp.float32),
            pltpu.SemaphoreType.DMA,
            pltpu.SemaphoreType.DMA,
            pltpu.SemaphoreType.DMA,
            pltpu.SemaphoreType.DMA,
        ],
    )(n_fea, src, dst)

    aggr2, cnt2 = pl.kernel(
        _phase_b,
        out_type=[
            jax.ShapeDtypeStruct((NC, NNP, HD), jnp.float32),
            jax.ShapeDtypeStruct((NC, NNP, 16), jnp.float32),
        ],
        mesh=_sc_mesh(),
        compiler_params=pltpu.CompilerParams(use_tc_tiling_on_sc=False),
        scratch_types=[
            pltpu.VMEM((PERT,), jnp.int32),
            pltpu.VMEM((K,), jnp.int32),
            pltpu.VMEM((K,), jnp.int32),
            pltpu.VMEM((K, HD), jnp.float32),
            pltpu.VMEM((K, HD), jnp.float32),
            pltpu.VMEM((K, 16), jnp.float32),
            pltpu.VMEM((40, HD), jnp.float32),
            pltpu.VMEM((40, 16), jnp.float32),
            pltpu.VMEM_SHARED((NNP, HD), jnp.float32),
            pltpu.VMEM_SHARED((NNP, 16), jnp.float32),
            pltpu.SemaphoreType.DMA,
            pltpu.SemaphoreType.DMA,
            pltpu.SemaphoreType.DMA,
            pltpu.SemaphoreType.DMA,
        ],
    )(hA, hB, src, dst)

    grid_spec = dict(
        grid=(_NT,),
        in_specs=[
            pl.BlockSpec((_T, HD), lambda i: (i, 0)),
            pl.BlockSpec((_T, HD), lambda i: (i, 0)),
            pl.BlockSpec((1, _T, HD),
                         lambda i: (0, jnp.minimum(i, _NZT - 1), 0)),
            pl.BlockSpec((1, _T, HD),
                         lambda i: (1, jnp.minimum(i, _NZT - 1), 0)),
            pl.BlockSpec((1, _T, 16),
                         lambda i: (0, jnp.minimum(i, _NZT - 1), 0)),
            pl.BlockSpec((1, _T, 16),
                         lambda i: (1, jnp.minimum(i, _NZT - 1), 0)),
            pl.BlockSpec((D, D), lambda i: (0, 0)),
            pl.BlockSpec((1, D), lambda i: (0, 0)),
            pl.BlockSpec((D, D), lambda i: (0, 0)),
            pl.BlockSpec((D, 64), lambda i: (0, 0)),
            pl.BlockSpec((1, 64), lambda i: (0, 0)),
            pl.BlockSpec((64, 32), lambda i: (0, 0)),
            pl.BlockSpec((1, 32), lambda i: (0, 0)),
            pl.BlockSpec((32, 2), lambda i: (0, 0)),
            pl.BlockSpec((1, 2), lambda i: (0, 0)),
        ],
        out_specs=pl.BlockSpec((_T, 2), lambda i: (i, 0)),
    )
    prob = pl.pallas_call(
        _tc_body,
        **grid_spec,
        out_shape=jax.ShapeDtypeStruct((E, 2), jnp.float32),
        scratch_shapes=[pltpu.VMEM((_T, D), jnp.float32)],
    )(hA, hB, aggr2, aggr2, cnt2, cnt2,
      W_sage_l, b_sage.reshape(1, D), W_sage_r,
      W1, b1.reshape(1, 64), W2, b2.reshape(1, 32), W3, b3.reshape(1, 2))

    label = jnp.concatenate([jnp.ones((EPOS,), jnp.int32),
                             jnp.zeros((EPOS,), jnp.int32)])
    return (prob, label)


# speedup vs baseline: 1.2485x; 1.2485x over previous
import jax
import jax.numpy as jnp
from jax import lax
from jax.experimental import pallas as pl
from jax.experimental.pallas import tpu as pltpu
from jax.experimental.pallas import tpu_sc as plsc

NN = 10000
D = 128
E = 320000
EPOS = 160000
NC = 2
NS = 16
NW = NC * NS
PERW = E // NW
PERT = E // NS
K = 80
NCHA = PERW // K
NCHB = PERT // K
HN = 5120
GR = HN
AH = 5248
RPT = AH // NS
NNP = 10240
CRPT = NNP // NS


def _sc_mesh():
    return plsc.VectorSubcoreMesh(core_axis_name="c", subcore_axis_name="s",
                                  num_cores=NC, num_subcores=NS)


def _mul_rows(ra, rb, oa):
    def rowfn(i, c2):
        for j in range(8):
            sl = pl.ds(j * 16, 16)
            oa[i, sl] = ra[i, sl] * rb[i, sl]
        return c2

    lax.fori_loop(0, K, rowfn, 0)


def _phase_a(nfea, src, dst, h_out, sall, dall, ra0, rb0, oa0, ra1, rb1, oa1,
             sg0, sg1, sw0, sw1):
    wid = lax.axis_index("c") * NS + lax.axis_index("s")
    base = wid * PERW

    pltpu.sync_copy(src.at[pl.ds(base, PERW)], sall)
    pltpu.sync_copy(dst.at[pl.ds(base, PERW)], dall)

    def gathers(ci, ra, rb, sg):
        off = ci * K
        pltpu.async_copy(nfea.at[sall.at[pl.ds(off, K)]], ra, sg)
        pltpu.async_copy(nfea.at[dall.at[pl.ds(off, K)]], rb, sg)

    gathers(0, ra0, rb0, sg0)
    gathers(1, ra1, rb1, sg1)

    def slot(ci, first, ra, rb, oa, sg, sw):
        out_slice = h_out.at[pl.ds(base + ci * K, K)]
        gd = pltpu.make_async_copy(nfea.at[sall.at[pl.ds(0, K)]], ra, sg)
        gd.wait()
        gd.wait()
        if not first:
            pltpu.make_async_copy(oa, out_slice, sw).wait()
        _mul_rows(ra, rb, oa)
        pltpu.async_copy(oa, out_slice, sw)

        if isinstance(ci, int):
            if ci + 2 < NCHA:
                gathers(ci + 2, ra, rb, sg)
        else:
            @pl.when(ci + 2 < NCHA)
            def _():
                gathers(ci + 2, ra, rb, sg)

    def pair(g, carry):
        slot(2 * g, False, ra0, rb0, oa0, sg0, sw0)
        slot(2 * g + 1, False, ra1, rb1, oa1, sg1, sw1)
        return carry

    slot(0, True, ra0, rb0, oa0, sg0, sw0)
    slot(1, True, ra1, rb1, oa1, sg1, sw1)
    lax.fori_loop(1, 62, pair, 0)
    slot(124, False, ra0, rb0, oa0, sg0, sw0)
    pltpu.make_async_copy(oa0, h_out.at[pl.ds(base, K)], sw0).wait()
    pltpu.make_async_copy(oa1, h_out.at[pl.ds(base, K)], sw1).wait()


def _phase_b(h, src, dst, aggr_out, cnt_out, sall, db0, db1, d20, d21, rw0,
             rw1, obuf, zb, zc, aggr_sp, cnt_sp, si0, si1, sg0, sg1):
    cid = lax.axis_index("c")
    sid = lax.axis_index("s")
    base = sid * PERT
    lo = cid * HN

    zero16 = jnp.zeros((16,), jnp.float32)
    e0 = jnp.where(lax.iota(jnp.int32, 16) == 0, 1.0, 0.0).astype(jnp.float32)

    def zrow(i, c2):
        for j in range(8):
            zb[i, pl.ds(j * 16, 16)] = zero16
        return c2

    lax.fori_loop(0, 40, zrow, 0)

    def zcrow(i, c2):
        zc[i, :] = zero16
        return c2

    lax.fori_loop(0, 40, zcrow, 0)

    def orow(i, c2):
        obuf[i, :] = e0
        return c2

    lax.fori_loop(0, K, orow, 0)

    for j in range(8):
        pltpu.sync_copy(zb, aggr_sp.at[pl.ds(sid * RPT + j * 40, 40)])
    pltpu.sync_copy(zb.at[pl.ds(0, 8)], aggr_sp.at[pl.ds(sid * RPT + 320, 8)])
    for j in range(16):
        pltpu.sync_copy(zc, cnt_sp.at[pl.ds(sid * CRPT + j * 40, 40)])

    pltpu.sync_copy(src.at[pl.ds(base, PERT)], sall)
    plsc.subcore_barrier()

    def issue(ci, db, rw, si, sg):
        off = ci * K
        pltpu.async_copy(dst.at[pl.ds(base + off, K)], db, si)
        pltpu.async_copy(h.at[sall.at[pl.ds(off, K)]], rw, sg)

    issue(0, db0, rw0, si0, sg0)
    issue(1, db1, rw1, si1, sg1)

    def slot(ci, even, db, d2, rw, si, sg):
        pltpu.make_async_copy(dst.at[pl.ds(base, K)], db, si).wait()

        def adj(v, c2):
            sl = pl.ds(v * 16, 16)
            d = db[sl] - lo
            ok = (d >= 0) & (d < HN)
            d2[sl] = jnp.where(ok, d, GR)
            return c2

        lax.fori_loop(0, K // 16, adj, 0)

        gd = pltpu.make_async_copy(h.at[sall.at[pl.ds(0, K)]], rw, sg)
        gd.wait()
        pltpu.sync_copy(rw, aggr_sp.at[d2], add=True)

        @pl.when(cid == (0 if even else 1))
        def _():
            pltpu.sync_copy(obuf, cnt_sp.at[db], add=True)

        @pl.when(ci + 2 < NCHB)
        def _():
            issue(ci + 2, db, rw, si, sg)

    def pair(g, carry):
        slot(2 * g, True, db0, d20, rw0, si0, sg0)
        slot(2 * g + 1, False, db1, d21, rw1, si1, sg1)
        return carry

    lax.fori_loop(0, 125, pair, 0)
    plsc.subcore_barrier()

    pltpu.sync_copy(aggr_sp.at[pl.ds(sid * RPT, RPT)],
                    aggr_out.at[cid, pl.ds(sid * RPT, RPT)])
    pltpu.sync_copy(cnt_sp.at[pl.ds(sid * CRPT, CRPT)],
                    cnt_out.at[cid, pl.ds(sid * CRPT, CRPT)])


_T = 2000
_NT = E // _T
_NZT = NN // _T


def _tc_body(h_ref, a_ref, c0_ref, c1_ref, wl, bl, wr, w1, b1, w2, b2,
             w3, b3, out_ref, z_scr):
    i = pl.program_id(0)

    @pl.when(i < _NZT)
    def _():
        cnt = c0_ref[0][:, 0:1] + c1_ref[0][:, 0:1]
        cnt1 = jnp.maximum(cnt, 1.0)
        z_scr[...] = jnp.dot(a_ref[...] / cnt1, wl[...],
                             preferred_element_type=jnp.float32)

    @pl.when(i == _NZT)
    def _():
        z_scr[...] = jnp.zeros((_T, D), jnp.float32)

    x = jnp.dot(h_ref[...].astype(jnp.bfloat16),
                wr[...].astype(jnp.bfloat16),
                preferred_element_type=jnp.float32)
    x = x + bl[...] + z_scr[...]
    x = jnp.maximum(x, 0.0)
    x = jnp.maximum(jnp.dot(x, w1[...], preferred_element_type=jnp.float32)
                    + b1[...], 0.0)
    x = jnp.maximum(jnp.dot(x, w2[...], preferred_element_type=jnp.float32)
                    + b2[...], 0.0)
    lg = jnp.dot(x, w3[...], preferred_element_type=jnp.float32) + b3[...]
    m = jnp.max(lg, axis=-1, keepdims=True)
    lse = m + jnp.log(jnp.sum(jnp.exp(lg - m), axis=-1, keepdims=True))
    out_ref[...] = lg - lse


def kernel(rna_f, protein_f, all_edges, W_sage_l, b_sage, W_sage_r,
           W1, b1, W2, b2, W3, b3):
    n_fea = jnp.concatenate([rna_f, protein_f], axis=0)
    edges = all_edges[::2].T
    neg = jax.random.randint(jax.random.key(42), (2, EPOS), 0, NN,
                             edges.dtype)
    ei = jnp.concatenate([edges, neg], axis=1)
    src = ei[0]
    dst = ei[1]

    h = pl.kernel(
        _phase_a,
        out_type=jax.ShapeDtypeStruct((E, D), jnp.float32),
        mesh=_sc_mesh(),
        compiler_params=pltpu.CompilerParams(use_tc_tiling_on_sc=False),
        scratch_types=[
            pltpu.VMEM((PERW,), jnp.int32),
            pltpu.VMEM((PERW,), jnp.int32),
            pltpu.VMEM((K, D), jnp.float32),
            pltpu.VMEM((K, D), jnp.float32),
            pltpu.VMEM((K, D), jnp.float32),
            pltpu.VMEM((K, D), jnp.float32),
            pltpu.VMEM((K, D), jnp.float32),
            pltpu.VMEM((K, D), jnp.float32),
            pltpu.SemaphoreType.DMA,
            pltpu.SemaphoreType.DMA,
            pltpu.SemaphoreType.DMA,
            pltpu.SemaphoreType.DMA,
        ],
    )(n_fea, src, dst)

    aggr2, cnt2 = pl.kernel(
        _phase_b,
        out_type=[
            jax.ShapeDtypeStruct((NC, AH, D), jnp.float32),
            jax.ShapeDtypeStruct((NC, NNP, 16), jnp.float32),
        ],
        mesh=_sc_mesh(),
        compiler_params=pltpu.CompilerParams(use_tc_tiling_on_sc=False),
        scratch_types=[
            pltpu.VMEM((PERT,), jnp.int32),
            pltpu.VMEM((K,), jnp.int32),
            pltpu.VMEM((K,), jnp.int32),
            pltpu.VMEM((K,), jnp.int32),
            pltpu.VMEM((K,), jnp.int32),
            pltpu.VMEM((K, D), jnp.float32),
            pltpu.VMEM((K, D), jnp.float32),
            pltpu.VMEM((K, 16), jnp.float32),
            pltpu.VMEM((40, D), jnp.float32),
            pltpu.VMEM((40, 16), jnp.float32),
            pltpu.VMEM_SHARED((AH, D), jnp.float32),
            pltpu.VMEM_SHARED((NNP, 16), jnp.float32),
            pltpu.SemaphoreType.DMA,
            pltpu.SemaphoreType.DMA,
            pltpu.SemaphoreType.DMA,
            pltpu.SemaphoreType.DMA,
        ],
    )(h, src, dst)

    aggr = jnp.concatenate([aggr2[0, :HN], aggr2[1, :NN - HN]], axis=0)

    grid_spec = dict(
        grid=(_NT,),
        in_specs=[
            pl.BlockSpec((_T, D), lambda i: (i, 0)),
            pl.BlockSpec((_T, D), lambda i: (jnp.minimum(i, _NZT - 1), 0)),
            pl.BlockSpec((1, _T, 16),
                         lambda i: (0, jnp.minimum(i, _NZT - 1), 0)),
            pl.BlockSpec((1, _T, 16),
                         lambda i: (1, jnp.minimum(i, _NZT - 1), 0)),
            pl.BlockSpec((D, D), lambda i: (0, 0)),
            pl.BlockSpec((1, D), lambda i: (0, 0)),
            pl.BlockSpec((D, D), lambda i: (0, 0)),
            pl.BlockSpec((D, 64), lambda i: (0, 0)),
            pl.BlockSpec((1, 64), lambda i: (0, 0)),
            pl.BlockSpec((64, 32), lambda i: (0, 0)),
            pl.BlockSpec((1, 32), lambda i: (0, 0)),
            pl.BlockSpec((32, 2), lambda i: (0, 0)),
            pl.BlockSpec((1, 2), lambda i: (0, 0)),
        ],
        out_specs=pl.BlockSpec((_T, 2), lambda i: (i, 0)),
    )
    prob = pl.pallas_call(
        _tc_body,
        **grid_spec,
        out_shape=jax.ShapeDtypeStruct((E, 2), jnp.float32),
        scratch_shapes=[pltpu.VMEM((_T, D), jnp.float32)],
    )(h, aggr, cnt2, cnt2,
      W_sage_l, b_sage.reshape(1, D), W_sage_r,
      W1, b1.reshape(1, 64), W2, b2.reshape(1, 32), W3, b3.reshape(1, 2))

    label = jnp.concatenate([jnp.ones((EPOS,), jnp.int32),
                             jnp.zeros((EPOS,), jnp.int32)])
    return (prob, label)


# speedup vs baseline: 1.2918x; 1.0347x over previous
import jax
import jax.numpy as jnp
from jax import lax
from jax.experimental import pallas as pl
from jax.experimental.pallas import tpu as pltpu
from jax.experimental.pallas import tpu_sc as plsc

NN = 10000
D = 128
E = 320000
EPOS = 160000
NC = 2
NS = 16
NW = NC * NS
PERW = E // NW
PERT = E // NS
K = 80
NCHA = PERW // K
NCHB = PERT // K
HN = 5120
GR = HN
AH = 5248
RPT = AH // NS
NNP = 10240
CRPT = NNP // NS


def _sc_mesh():
    return plsc.VectorSubcoreMesh(core_axis_name="c", subcore_axis_name="s",
                                  num_cores=NC, num_subcores=NS)


def _mul_rows(ra, rb, oa):
    def rowfn(i, c2):
        r0 = 2 * i
        r1 = 2 * i + 1
        for j in range(8):
            sl = pl.ds(j * 16, 16)
            oa[r0, sl] = ra[r0, sl] * rb[r0, sl]
        for j in range(8):
            sl = pl.ds(j * 16, 16)
            oa[r1, sl] = ra[r1, sl] * rb[r1, sl]
        return c2

    lax.fori_loop(0, K // 2, rowfn, 0)


def _phase_a(nfea, ei, h_out, sall, dall, ra0, rb0, oa0, ra1, rb1, oa1,
             sg0, sg1, sw0, sw1):
    wid = lax.axis_index("c") * NS + lax.axis_index("s")
    base = wid * PERW

    pltpu.sync_copy(ei.at[0, pl.ds(base, PERW)], sall)
    pltpu.sync_copy(ei.at[1, pl.ds(base, PERW)], dall)

    def gathers(ci, ra, rb, sg):
        off = ci * K
        pltpu.async_copy(nfea.at[sall.at[pl.ds(off, K)]], ra, sg)
        pltpu.async_copy(nfea.at[dall.at[pl.ds(off, K)]], rb, sg)

    gathers(0, ra0, rb0, sg0)
    gathers(1, ra1, rb1, sg1)

    def slot(ci, first, ra, rb, oa, sg, sw):
        out_slice = h_out.at[pl.ds(base + ci * K, K)]
        gd = pltpu.make_async_copy(nfea.at[sall.at[pl.ds(0, K)]], ra, sg)
        gd.wait()
        gd.wait()
        if not first:
            pltpu.make_async_copy(oa, out_slice, sw).wait()
        _mul_rows(ra, rb, oa)
        pltpu.async_copy(oa, out_slice, sw)

        if isinstance(ci, int):
            if ci + 2 < NCHA:
                gathers(ci + 2, ra, rb, sg)
        else:
            @pl.when(ci + 2 < NCHA)
            def _():
                gathers(ci + 2, ra, rb, sg)

    def pair(g, carry):
        slot(2 * g, False, ra0, rb0, oa0, sg0, sw0)
        slot(2 * g + 1, False, ra1, rb1, oa1, sg1, sw1)
        return carry

    slot(0, True, ra0, rb0, oa0, sg0, sw0)
    slot(1, True, ra1, rb1, oa1, sg1, sw1)
    lax.fori_loop(1, 62, pair, 0)
    slot(124, False, ra0, rb0, oa0, sg0, sw0)
    pltpu.make_async_copy(oa0, h_out.at[pl.ds(base, K)], sw0).wait()
    pltpu.make_async_copy(oa1, h_out.at[pl.ds(base, K)], sw1).wait()


def _phase_b(h, ei, aggr_out, cnt_out, sall, db, d2, rw, obuf,
             zb, zc, aggr_sp, cnt_sp, di0, di1, di2, di3,
             sg0, sg1, sg2, sg3, ss0, ss1, ss2, ss3):
    cid = lax.axis_index("c")
    sid = lax.axis_index("s")
    base = sid * PERT
    lo = cid * HN
    di = [di0, di1, di2, di3]
    sg = [sg0, sg1, sg2, sg3]
    ss = [ss0, ss1, ss2, ss3]

    zero16 = jnp.zeros((16,), jnp.float32)
    e0 = jnp.where(lax.iota(jnp.int32, 16) == 0, 1.0, 0.0).astype(jnp.float32)

    def zrow(i, c2):
        for j in range(8):
            zb[i, pl.ds(j * 16, 16)] = zero16
        return c2

    lax.fori_loop(0, 40, zrow, 0)

    def zcrow(i, c2):
        zc[i, :] = zero16
        return c2

    lax.fori_loop(0, 40, zcrow, 0)

    def orow(i, c2):
        obuf[i, :] = e0
        return c2

    lax.fori_loop(0, K, orow, 0)

    for j in range(8):
        pltpu.sync_copy(zb, aggr_sp.at[pl.ds(sid * RPT + j * 40, 40)])
    pltpu.sync_copy(zb.at[pl.ds(0, 8)], aggr_sp.at[pl.ds(sid * RPT + 320, 8)])
    for j in range(16):
        pltpu.sync_copy(zc, cnt_sp.at[pl.ds(sid * CRPT + j * 40, 40)])

    pltpu.sync_copy(ei.at[0, pl.ds(base, PERT)], sall)
    plsc.subcore_barrier()

    def issue(ci, q):
        off = ci * K
        pltpu.async_copy(ei.at[1, pl.ds(base + off, K)], db.at[q], di[q])
        pltpu.async_copy(h.at[sall.at[pl.ds(off, K)]], rw.at[q], sg[q])

    issue(0, 0)
    issue(1, 1)

    def slot(ci, q, even, drain_prev):
        pltpu.make_async_copy(ei.at[1, pl.ds(base, K)], db.at[q],
                              di[q]).wait()

        def adj(v, c2):
            sl = pl.ds(v * 16, 16)
            d = db[q, sl] - lo
            ok = (d >= 0) & (d < HN)
            d2[q, sl] = jnp.where(ok, d, GR)
            return c2

        lax.fori_loop(0, K // 16, adj, 0)

        pltpu.make_async_copy(h.at[sall.at[pl.ds(0, K)]], rw.at[q],
                              sg[q]).wait()
        pltpu.async_copy(rw.at[q], aggr_sp.at[d2.at[q]], ss[q], add=True)

        @pl.when(cid == (0 if even else 1))
        def _():
            pltpu.sync_copy(obuf, cnt_sp.at[db.at[q]], add=True)

        q2 = (q + 2) % 4
        if drain_prev:
            pltpu.make_async_copy(rw.at[q2], aggr_sp.at[d2.at[q2]],
                                  ss[q2]).wait()

        def _issue_next():
            issue(ci + 2, q2)

        if isinstance(ci, int):
            if ci + 2 < NCHB:
                _issue_next()
        else:
            @pl.when(ci + 2 < NCHB)
            def _():
                _issue_next()

    def quad(g, carry):
        slot(4 * g, 0, True, True)
        slot(4 * g + 1, 1, False, True)
        slot(4 * g + 2, 2, True, True)
        slot(4 * g + 3, 3, False, True)
        return carry

    slot(0, 0, True, False)
    slot(1, 1, False, False)
    slot(2, 2, True, True)
    slot(3, 3, False, True)
    lax.fori_loop(1, 62, quad, 0)
    slot(248, 0, True, True)
    slot(249, 1, False, True)
    for q in range(2):
        pltpu.make_async_copy(rw.at[q], aggr_sp.at[d2.at[q]], ss[q]).wait()
    plsc.subcore_barrier()

    pltpu.sync_copy(aggr_sp.at[pl.ds(sid * RPT, RPT)],
                    aggr_out.at[cid, pl.ds(sid * RPT, RPT)])
    pltpu.sync_copy(cnt_sp.at[pl.ds(sid * CRPT, CRPT)],
                    cnt_out.at[cid, pl.ds(sid * CRPT, CRPT)])


_T = 2000
_NT = E // _T
_NZT = NN // _T


def _tc_body(h_ref, a_ref, c0_ref, c1_ref, wl, bl, wr, w1, b1, w2, b2,
             w3, b3, out_ref, z_scr):
    i = pl.program_id(0)

    @pl.when(i < _NZT)
    def _():
        cnt = c0_ref[0][:, 0:1] + c1_ref[0][:, 0:1]
        cnt1 = jnp.maximum(cnt, 1.0)
        z_scr[...] = jnp.dot(a_ref[...] / cnt1, wl[...],
                             preferred_element_type=jnp.float32)

    @pl.when(i >= _NZT)
    def _():
        z_scr[...] = jnp.zeros((_T, D), jnp.float32)

    x = jnp.dot(h_ref[...], wr[...], preferred_element_type=jnp.float32)
    x = x + bl[...] + z_scr[...]
    x = jnp.maximum(x, 0.0)
    x = jnp.maximum(jnp.dot(x, w1[...], preferred_element_type=jnp.float32)
                    + b1[...], 0.0)
    x = jnp.maximum(jnp.dot(x, w2[...], preferred_element_type=jnp.float32)
                    + b2[...], 0.0)
    lg = jnp.dot(x, w3[...], preferred_element_type=jnp.float32) + b3[...]
    m = jnp.max(lg, axis=-1, keepdims=True)
    lse = m + jnp.log(jnp.sum(jnp.exp(lg - m), axis=-1, keepdims=True))
    out_ref[...] = lg - lse


def kernel(rna_f, protein_f, all_edges, W_sage_l, b_sage, W_sage_r,
           W1, b1, W2, b2, W3, b3):
    n_fea = jnp.concatenate([rna_f, protein_f], axis=0)
    edges = all_edges[::2].T
    neg = jax.random.randint(jax.random.key(42), (2, EPOS), 0, NN,
                             edges.dtype)
    ei = jnp.concatenate([edges, neg], axis=1)

    h = pl.kernel(
        _phase_a,
        out_type=jax.ShapeDtypeStruct((E, D), jnp.float32),
        mesh=_sc_mesh(),
        compiler_params=pltpu.CompilerParams(use_tc_tiling_on_sc=False),
        scratch_types=[
            pltpu.VMEM((PERW,), jnp.int32),
            pltpu.VMEM((PERW,), jnp.int32),
            pltpu.VMEM((K, D), jnp.float32),
            pltpu.VMEM((K, D), jnp.float32),
            pltpu.VMEM((K, D), jnp.float32),
            pltpu.VMEM((K, D), jnp.float32),
            pltpu.VMEM((K, D), jnp.float32),
            pltpu.VMEM((K, D), jnp.float32),
            pltpu.SemaphoreType.DMA,
            pltpu.SemaphoreType.DMA,
            pltpu.SemaphoreType.DMA,
            pltpu.SemaphoreType.DMA,
        ],
    )(n_fea, ei)

    aggr2, cnt2 = pl.kernel(
        _phase_b,
        out_type=[
            jax.ShapeDtypeStruct((NC, AH, D), jnp.float32),
            jax.ShapeDtypeStruct((NC, NNP, 16), jnp.float32),
        ],
        mesh=_sc_mesh(),
        compiler_params=pltpu.CompilerParams(use_tc_tiling_on_sc=False),
        scratch_types=[
            pltpu.VMEM((PERT,), jnp.int32),
            pltpu.VMEM((4, K), jnp.int32),
            pltpu.VMEM((4, K), jnp.int32),
            pltpu.VMEM((4, K, D), jnp.float32),
            pltpu.VMEM((K, 16), jnp.float32),
            pltpu.VMEM((40, D), jnp.float32),
            pltpu.VMEM((40, 16), jnp.float32),
            pltpu.VMEM_SHARED((AH, D), jnp.float32),
            pltpu.VMEM_SHARED((NNP, 16), jnp.float32),
        ] + [pltpu.SemaphoreType.DMA] * 12,
    )(h, ei)

    aggr = jnp.concatenate([aggr2[0, :HN], aggr2[1, :NN - HN]], axis=0)

    grid_spec = dict(
        grid=(_NT,),
        in_specs=[
            pl.BlockSpec((_T, D), lambda i: (i, 0)),
            pl.BlockSpec((_T, D), lambda i: (jnp.minimum(i, _NZT - 1), 0)),
            pl.BlockSpec((1, _T, 16),
                         lambda i: (0, jnp.minimum(i, _NZT - 1), 0)),
            pl.BlockSpec((1, _T, 16),
                         lambda i: (1, jnp.minimum(i, _NZT - 1), 0)),
            pl.BlockSpec((D, D), lambda i: (0, 0)),
            pl.BlockSpec((1, D), lambda i: (0, 0)),
            pl.BlockSpec((D, D), lambda i: (0, 0)),
            pl.BlockSpec((D, 64), lambda i: (0, 0)),
            pl.BlockSpec((1, 64), lambda i: (0, 0)),
            pl.BlockSpec((64, 32), lambda i: (0, 0)),
            pl.BlockSpec((1, 32), lambda i: (0, 0)),
            pl.BlockSpec((32, 2), lambda i: (0, 0)),
            pl.BlockSpec((1, 2), lambda i: (0, 0)),
        ],
        out_specs=pl.BlockSpec((_T, 2), lambda i: (i, 0)),
    )
    prob = pl.pallas_call(
        _tc_body,
        **grid_spec,
        out_shape=jax.ShapeDtypeStruct((E, 2), jnp.float32),
        scratch_shapes=[pltpu.VMEM((_T, D), jnp.float32)],
    )(h, aggr, cnt2, cnt2,
      W_sage_l, b_sage.reshape(1, D), W_sage_r,
      W1, b1.reshape(1, 64), W2, b2.reshape(1, 32), W3, b3.reshape(1, 2))

    label = jnp.concatenate([jnp.ones((EPOS,), jnp.int32),
                             jnp.zeros((EPOS,), jnp.int32)])
    return (prob, label)
